# baseline (speedup 1.0000x reference)
import jax
import jax.numpy as jnp
from jax.experimental import pallas as pl
from jax.experimental.pallas import tpu as pltpu


def _body(x_ref, pos_ref, g_ref, b_ref, o_ref):
    x = x_ref[...]
    p = pos_ref[...]
    y = x + p
    mu = jnp.mean(y, axis=-1, keepdims=True)
    var = jnp.mean(y * y, axis=-1, keepdims=True) - mu * mu
    xhat = (y - mu) * jax.lax.rsqrt(var + 1e-12)
    o_ref[...] = xhat * g_ref[...] + b_ref[...]


def kernel(input_embeddings, pos_table, gamma, beta):
    B, S, D = input_embeddings.shape
    R_BLK = 2048
    n_pos = S // R_BLK
    x2 = input_embeddings.reshape(B * S, D)
    g2 = gamma.reshape(1, D)
    b2 = beta.reshape(1, D)
    out = pl.pallas_call(
        _body,
        grid=(n_pos, B),
        in_specs=[
            pl.BlockSpec((R_BLK, D), lambda j, b: (b * n_pos + j, 0)),
            pl.BlockSpec((R_BLK, D), lambda j, b: (j, 0)),
            pl.BlockSpec((1, D), lambda j, b: (0, 0)),
            pl.BlockSpec((1, D), lambda j, b: (0, 0)),
        ],
        out_specs=pl.BlockSpec((R_BLK, D), lambda j, b: (b * n_pos + j, 0)),
        out_shape=jax.ShapeDtypeStruct((B * S, D), jnp.float32),
        compiler_params=pltpu.CompilerParams(
            vmem_limit_bytes=100 * 1024 * 1024,
        ),
    )(x2, pos_table, g2, b2)
    return out.reshape(B, S, D)


# speedup vs baseline: 1.0447x; 1.0447x over previous
import jax
import jax.numpy as jnp
from jax.experimental import pallas as pl
from jax.experimental.pallas import tpu as pltpu


def _body(x_ref, pos_ref, g_ref, b_ref, o_ref):
    x = x_ref[...]
    p = pos_ref[...]
    y = x + p[None, :, :]
    mu = jnp.mean(y, axis=-1, keepdims=True)
    var = jnp.mean(y * y, axis=-1, keepdims=True) - mu * mu
    xhat = (y - mu) * jax.lax.rsqrt(var + 1e-12)
    o_ref[...] = xhat * g_ref[...] + b_ref[...]


def kernel(input_embeddings, pos_table, gamma, beta):
    B, S, D = input_embeddings.shape
    S_BLK = 512
    grid = (S // S_BLK,)
    g2 = gamma.reshape(1, 1, D)
    b2 = beta.reshape(1, 1, D)
    return pl.pallas_call(
        _body,
        grid=grid,
        in_specs=[
            pl.BlockSpec((B, S_BLK, D), lambda i: (0, i, 0)),
            pl.BlockSpec((S_BLK, D), lambda i: (i, 0)),
            pl.BlockSpec((1, 1, D), lambda i: (0, 0, 0)),
            pl.BlockSpec((1, 1, D), lambda i: (0, 0, 0)),
        ],
        out_specs=pl.BlockSpec((B, S_BLK, D), lambda i: (0, i, 0)),
        out_shape=jax.ShapeDtypeStruct((B, S, D), jnp.float32),
        compiler_params=pltpu.CompilerParams(
            dimension_semantics=("parallel",),
            vmem_limit_bytes=100 * 1024 * 1024,
        ),
    )(input_embeddings, pos_table, g2, b2)


# speedup vs baseline: 1.0954x; 1.0486x over previous
import jax
import jax.numpy as jnp
from jax import lax
from jax.experimental import pallas as pl
from jax.experimental.pallas import tpu as pltpu

_NBUF = 4
_C = 256


def _body(x_hbm, pos_hbm, g_ref, b_ref, o_hbm,
          xbuf, pbuf, obuf, insem, psem, outsem):
    n_steps = x_hbm.shape[1] // _C

    def in_copies(i):
        slot = i % _NBUF
        cx = pltpu.make_async_copy(
            x_hbm.at[:, pl.ds(i * _C, _C), :], xbuf.at[slot], insem.at[slot])
        cp = pltpu.make_async_copy(
            pos_hbm.at[pl.ds(i * _C, _C), :], pbuf.at[slot], psem.at[slot])
        return cx, cp

    def out_copy(i):
        slot = i % _NBUF
        return pltpu.make_async_copy(
            obuf.at[slot], o_hbm.at[:, pl.ds(i * _C, _C), :], outsem.at[slot])

    for i in range(_NBUF - 1):
        cx, cp = in_copies(i)
        cx.start()
        cp.start()

    g = g_ref[...]
    b = b_ref[...]

    for i in range(n_steps):
        slot = i % _NBUF
        cx, cp = in_copies(i)
        cx.wait()
        cp.wait()
        if i >= _NBUF:
            out_copy(i - _NBUF).wait()

        y = xbuf[slot] + pbuf[slot][None, :, :]
        mu = jnp.mean(y, axis=-1, keepdims=True)
        var = jnp.mean(y * y, axis=-1, keepdims=True) - mu * mu
        xhat = (y - mu) * lax.rsqrt(var + 1e-12)
        obuf[slot] = xhat * g + b

        out_copy(i).start()

        nxt = i + _NBUF - 1
        if nxt < n_steps:
            cx2, cp2 = in_copies(nxt)
            cx2.start()
            cp2.start()

    for i in range(n_steps - _NBUF, n_steps):
        out_copy(i).wait()


def kernel(input_embeddings, pos_table, gamma, beta):
    B, S, D = input_embeddings.shape
    g2 = gamma.reshape(1, 1, D)
    b2 = beta.reshape(1, 1, D)
    return pl.pallas_call(
        _body,
        in_specs=[
            pl.BlockSpec(memory_space=pltpu.MemorySpace.HBM),
            pl.BlockSpec(memory_space=pltpu.MemorySpace.HBM),
            pl.BlockSpec((1, 1, D), lambda: (0, 0, 0)),
            pl.BlockSpec((1, 1, D), lambda: (0, 0, 0)),
        ],
        out_specs=pl.BlockSpec(memory_space=pltpu.MemorySpace.HBM),
        out_shape=jax.ShapeDtypeStruct((B, S, D), jnp.float32),
        scratch_shapes=[
            pltpu.VMEM((_NBUF, B, _C, D), jnp.float32),
            pltpu.VMEM((_NBUF, _C, D), jnp.float32),
            pltpu.VMEM((_NBUF, B, _C, D), jnp.float32),
            pltpu.SemaphoreType.DMA((_NBUF,)),
            pltpu.SemaphoreType.DMA((_NBUF,)),
            pltpu.SemaphoreType.DMA((_NBUF,)),
        ],
        compiler_params=pltpu.CompilerParams(
            vmem_limit_bytes=100 * 1024 * 1024,
        ),
    )(input_embeddings, pos_table, g2, b2)
